# Initial kernel scaffold; baseline (speedup 1.0000x reference)
import jax
import jax.numpy as jnp
from jax.experimental import pallas as pl


def kernel(x, edge_index, Wl1, Wr1, b1, Wl2, Wr2, b2, Wl3, Wr3, b3, Wl4, Wr4, b4):
    raise NotImplementedError("write your pallas kernel here")


# speedup vs baseline: 5.6121x; 5.6121x over previous
import functools

import jax
import jax.numpy as jnp
from jax import lax
from jax.experimental import pallas as pl
from jax.experimental.pallas import tpu as pltpu
from jax.experimental.pallas import tpu_sc as plsc

_NUM_WORKERS = 32
_NUM_CORES = 2
_NUM_SUBCORES = 16
_K = 128


def _mesh():
    return plsc.VectorSubcoreMesh(core_axis_name="c", subcore_axis_name="s")


@functools.lru_cache(maxsize=None)
def _make_agg(NP: int, d: int, EP: int):
    e_per_w = EP // _NUM_WORKERS
    nchunks = e_per_w // _K
    rpt = NP // _NUM_SUBCORES

    @functools.partial(
        pl.kernel,
        out_type=jax.ShapeDtypeStruct((_NUM_CORES, NP, d), jnp.float32),
        mesh=_mesh(),
        scratch_types=[
            pltpu.VMEM((_K,), jnp.int32),
            pltpu.VMEM((_K,), jnp.int32),
            pltpu.VMEM((_K, d), jnp.float32),
            pltpu.VMEM_SHARED((NP, d), jnp.float32),
            pltpu.SemaphoreType.DMA,
        ],
    )
    def k(table, src, dst, zeros, out, src_v, dst_v, rows_v, agg_sh, sem):
        c = lax.axis_index("c")
        s = lax.axis_index("s")
        w = s * _NUM_CORES + c
        pltpu.sync_copy(zeros.at[pl.ds(s * rpt, rpt)],
                        agg_sh.at[pl.ds(s * rpt, rpt)])
        plsc.subcore_barrier()

        def body(j, carry):
            base = w * e_per_w + j * _K
            pltpu.sync_copy(src.at[pl.ds(base, _K)], src_v)
            pltpu.sync_copy(dst.at[pl.ds(base, _K)], dst_v)
            pltpu.async_copy(table.at[src_v], rows_v, sem).wait()
            pltpu.sync_copy(rows_v, agg_sh.at[dst_v], add=True)
            return carry

        lax.fori_loop(0, nchunks, body, 0)
        plsc.subcore_barrier()
        pltpu.sync_copy(agg_sh.at[pl.ds(s * rpt, rpt)],
                        out.at[c, pl.ds(s * rpt, rpt)])

    return k


@functools.lru_cache(maxsize=None)
def _make_deg(NP: int, EP: int):
    e_per_w = EP // _NUM_WORKERS
    nchunks = e_per_w // _K
    rpt = NP // _NUM_SUBCORES

    @functools.partial(
        pl.kernel,
        out_type=jax.ShapeDtypeStruct((_NUM_CORES, NP, 128), jnp.float32),
        mesh=_mesh(),
        scratch_types=[
            pltpu.VMEM((_K,), jnp.int32),
            pltpu.VMEM((_K, 128), jnp.float32),
            pltpu.VMEM_SHARED((NP, 128), jnp.float32),
        ],
    )
    def k(dst, zeros, ones, out, dst_v, ones_v, deg_sh):
        c = lax.axis_index("c")
        s = lax.axis_index("s")
        w = s * _NUM_CORES + c
        pltpu.sync_copy(zeros.at[pl.ds(s * rpt, rpt)],
                        deg_sh.at[pl.ds(s * rpt, rpt)])
        pltpu.sync_copy(ones, ones_v)
        plsc.subcore_barrier()

        def body(j, carry):
            base = w * e_per_w + j * _K
            pltpu.sync_copy(dst.at[pl.ds(base, _K)], dst_v)
            pltpu.sync_copy(ones_v, deg_sh.at[dst_v], add=True)
            return carry

        lax.fori_loop(0, nchunks, body, 0)
        plsc.subcore_barrier()
        pltpu.sync_copy(deg_sh.at[pl.ds(s * rpt, rpt)],
                        out.at[c, pl.ds(s * rpt, rpt)])

    return k


@functools.lru_cache(maxsize=None)
def _make_dense(NP: int, R: int, da: int, din: int, dout: int, dnext: int,
                pre: bool):
    grid = NP // R

    def body(*refs):
        it = iter(refs)
        a0, a1, degp, h = next(it), next(it), next(it), next(it)
        wl = next(it) if pre else None
        wr, b = next(it), next(it)
        wn = next(it) if dnext > 0 else None
        out = next(it)
        pout = next(it) if dnext > 0 else None
        dsum = degp[0] + degp[1]
        deg = jnp.maximum(dsum[:, 0:1], 1.0)
        mean = (a0[...] + a1[...]) / deg
        if pre:
            acc = jnp.dot(mean, wl[...], preferred_element_type=jnp.float32)
        else:
            acc = mean
        acc = acc + jnp.dot(h[...], wr[...],
                            preferred_element_type=jnp.float32)
        res = jnp.maximum(acc + b[...], 0.0)
        out[...] = res
        if dnext > 0:
            pout[...] = jnp.dot(res, wn[...],
                                preferred_element_type=jnp.float32)

    in_specs = [
        pl.BlockSpec((R, da), lambda i: (i, 0)),
        pl.BlockSpec((R, da), lambda i: (i, 0)),
        pl.BlockSpec((2, R, 128), lambda i: (0, i, 0)),
        pl.BlockSpec((R, din), lambda i: (i, 0)),
    ]
    if pre:
        in_specs.append(pl.BlockSpec((da, dout), lambda i: (0, 0)))
    in_specs += [
        pl.BlockSpec((din, dout), lambda i: (0, 0)),
        pl.BlockSpec((1, dout), lambda i: (0, 0)),
    ]
    out_shape = [jax.ShapeDtypeStruct((NP, dout), jnp.float32)]
    out_specs = [pl.BlockSpec((R, dout), lambda i: (i, 0))]
    if dnext > 0:
        in_specs.append(pl.BlockSpec((dout, dnext), lambda i: (0, 0)))
        out_shape.append(jax.ShapeDtypeStruct((NP, dnext), jnp.float32))
        out_specs.append(pl.BlockSpec((R, dnext), lambda i: (i, 0)))

    return pl.pallas_call(
        body,
        grid=(grid,),
        in_specs=in_specs,
        out_specs=out_specs,
        out_shape=out_shape,
    )


@functools.lru_cache(maxsize=None)
def _make_final(NP: int, N: int, R: int, da: int, dain: int, dh: int):
    grid = NP // R

    def body(a0, a1, degp, h5, wr4, b4, pooled, logp, acc1, acc2):
        i = pl.program_id(0)

        @pl.when(i == 0)
        def _init():
            acc1[...] = jnp.zeros_like(acc1)
            acc2[...] = jnp.zeros_like(acc2)

        dsum = degp[0] + degp[1]
        deg = jnp.maximum(dsum[:, 0:1], 1.0)
        row = i * R + lax.broadcasted_iota(jnp.int32, (R, 1), 0)
        mask = (row < N).astype(jnp.float32)
        asum = (a0[...] + a1[...])[:, 0:da]
        contrib = asum / deg * mask
        acc1[...] += jnp.sum(contrib, axis=0, keepdims=True)
        acc2[...] += jnp.sum(h5[...] * mask, axis=0, keepdims=True)

        @pl.when(i == grid - 1)
        def _emit():
            s1 = acc1[...]
            s2 = acc2[...]
            p = (s1 + jnp.dot(s2, wr4[...],
                              preferred_element_type=jnp.float32)) / N
            p = p + b4[...]
            m = jnp.max(p, axis=1, keepdims=True)
            lse = jnp.log(jnp.sum(jnp.exp(p - m), axis=1, keepdims=True)) + m
            pooled[...] = p
            logp[...] = p - lse

    return pl.pallas_call(
        body,
        grid=(grid,),
        in_specs=[
            pl.BlockSpec((R, dain), lambda i: (i, 0)),
            pl.BlockSpec((R, dain), lambda i: (i, 0)),
            pl.BlockSpec((2, R, 128), lambda i: (0, i, 0)),
            pl.BlockSpec((R, dh), lambda i: (i, 0)),
            pl.BlockSpec((dh, da), lambda i: (0, 0)),
            pl.BlockSpec((1, da), lambda i: (0, 0)),
        ],
        out_specs=[
            pl.BlockSpec((1, da), lambda i: (0, 0)),
            pl.BlockSpec((1, da), lambda i: (0, 0)),
        ],
        out_shape=[
            jax.ShapeDtypeStruct((1, da), jnp.float32),
            jax.ShapeDtypeStruct((1, da), jnp.float32),
        ],
        scratch_shapes=[
            pltpu.VMEM((1, da), jnp.float32),
            pltpu.VMEM((1, dh), jnp.float32),
        ],
    )


def _round_up(v: int, m: int) -> int:
    return (v + m - 1) // m * m


def kernel(x, edge_index, Wl1, Wr1, b1, Wl2, Wr2, b2, Wl3, Wr3, b3,
           Wl4, Wr4, b4):
    N, D = x.shape
    E = edge_index.shape[1]
    H2 = Wl1.shape[1]
    H = Wl2.shape[1]
    C = Wl4.shape[1]

    NP = _round_up(N, _NUM_SUBCORES * 8)
    EP = _round_up(E, _NUM_WORKERS * _K)
    R = NP // 8

    pad = EP - E
    pad_idx = (N + (NP - N - 16)
               + (jnp.arange(pad, dtype=jnp.int32) % 16))
    src_p = jnp.concatenate([edge_index[0], pad_idx])
    dst_p = jnp.concatenate([edge_index[1], pad_idx])
    x_p = jnp.pad(x, ((0, NP - N), (0, 0)))

    zeros_d = jnp.zeros((NP, H), jnp.float32)
    ones_d = jnp.ones((_K, 128), jnp.float32)

    agg_h = _make_agg(NP, H, EP)
    deg_k = _make_deg(NP, EP)

    degp = deg_k(dst_p, zeros_d, ones_d)

    b1r = b1.reshape(1, -1)
    b2r = b2.reshape(1, -1)
    b3r = b3.reshape(1, -1)
    b4r = b4.reshape(1, -1)

    a1p = agg_h(x_p, src_p, dst_p, zeros_d)
    h1, p2 = _make_dense(NP, R, D, D, H2, H, True)(
        a1p[0], a1p[1], degp, x_p, Wl1, Wr1, b1r, Wl2)

    a2p = agg_h(p2, src_p, dst_p, zeros_d)
    h2 = _make_dense(NP, R, H, H2, H, 0, False)(
        a2p[0], a2p[1], degp, h1, Wr2, b2r)[0]

    a3p = agg_h(h2, src_p, dst_p, zeros_d)
    h3 = _make_dense(NP, R, H, H, H, 0, True)(
        a3p[0], a3p[1], degp, h2, Wl3, Wr3, b3r)[0]
    a4p = agg_h(h3, src_p, dst_p, zeros_d)
    h4 = _make_dense(NP, R, H, H, H, 0, True)(
        a4p[0], a4p[1], degp, h3, Wl3, Wr3, b3r)[0]
    a5p = agg_h(h4, src_p, dst_p, zeros_d)
    Wl4p = jnp.pad(Wl4, ((0, 0), (0, H - C)))
    h5, p6 = _make_dense(NP, R, H, H, H, H, True)(
        a5p[0], a5p[1], degp, h4, Wl3, Wr3, b3r, Wl4p)

    a6p = agg_h(p6, src_p, dst_p, zeros_d)
    pooled, logp = _make_final(NP, N, R, C, H, H)(
        a6p[0], a6p[1], degp, h5, Wr4, b4r)
    return (pooled, logp)


# speedup vs baseline: 8.6109x; 1.5343x over previous
import functools

import jax
import jax.numpy as jnp
from jax import lax
from jax.experimental import pallas as pl
from jax.experimental.pallas import tpu as pltpu
from jax.experimental.pallas import tpu_sc as plsc

_NUM_WORKERS = 32
_NUM_CORES = 2
_NUM_SUBCORES = 16
_K = 128


def _mesh():
    return plsc.VectorSubcoreMesh(core_axis_name="c", subcore_axis_name="s")


@functools.lru_cache(maxsize=None)
def _make_agg(NP: int, d: int, EP: int):
    e_per_w = EP // _NUM_WORKERS
    nchunks = e_per_w // _K
    assert nchunks % 2 == 0
    rpt = NP // _NUM_SUBCORES

    @functools.partial(
        pl.kernel,
        out_type=jax.ShapeDtypeStruct((_NUM_CORES, NP, d), jnp.float32),
        mesh=_mesh(),
        scratch_types=[
            pltpu.VMEM((2 * _K,), jnp.int32),
            pltpu.VMEM((2 * _K,), jnp.int32),
            pltpu.VMEM((2, _K, d), jnp.float32),
            pltpu.VMEM_SHARED((NP, d), jnp.float32),
            pltpu.SemaphoreType.DMA,
            pltpu.SemaphoreType.DMA,
        ],
    )
    def k(table, src, dst, zeros, out, src_v, dst_v, rows_v, agg_sh,
          sem0, sem1):
        c = lax.axis_index("c")
        s = lax.axis_index("s")
        w = s * _NUM_CORES + c
        pltpu.sync_copy(zeros.at[pl.ds(s * rpt, rpt)],
                        agg_sh.at[pl.ds(s * rpt, rpt)])
        plsc.subcore_barrier()
        sems = (sem0, sem1)

        def fetch_idx(j, sl):
            pltpu.sync_copy(src.at[w, pl.ds(j * _K, _K)],
                            src_v.at[pl.ds(sl * _K, _K)])
            pltpu.sync_copy(dst.at[w, pl.ds(j * _K, _K)],
                            dst_v.at[pl.ds(sl * _K, _K)])

        def gather(sl):
            return pltpu.async_copy(
                table.at[src_v.at[pl.ds(sl * _K, _K)]], rows_v.at[sl],
                sems[sl])

        def drain_scat(sl):
            pltpu.make_async_copy(
                table.at[src_v.at[pl.ds(sl * _K, _K)]], rows_v.at[sl],
                sems[sl]).wait()
            pltpu.sync_copy(rows_v.at[sl],
                            agg_sh.at[dst_v.at[pl.ds(sl * _K, _K)]],
                            add=True)

        fetch_idx(0, 0)
        gather(0)

        def body(i, carry):
            j1 = 2 * i + 1
            fetch_idx(j1, 1)
            gather(1)
            drain_scat(0)

            @pl.when(j1 + 1 < nchunks)
            def _():
                fetch_idx(j1 + 1, 0)
                gather(0)

            drain_scat(1)
            return carry

        lax.fori_loop(0, nchunks // 2, body, 0)
        plsc.subcore_barrier()
        pltpu.sync_copy(agg_sh.at[pl.ds(s * rpt, rpt)],
                        out.at[c, pl.ds(s * rpt, rpt)])

    return k


@functools.lru_cache(maxsize=None)
def _make_deg(NP: int, EP: int):
    e_per_w = EP // _NUM_WORKERS
    nchunks = e_per_w // _K
    rpt = NP // _NUM_SUBCORES

    @functools.partial(
        pl.kernel,
        out_type=jax.ShapeDtypeStruct((_NUM_CORES, NP, 128), jnp.float32),
        mesh=_mesh(),
        scratch_types=[
            pltpu.VMEM((2 * _K,), jnp.int32),
            pltpu.VMEM((_K, 128), jnp.float32),
            pltpu.VMEM_SHARED((NP, 128), jnp.float32),
            pltpu.SemaphoreType.DMA,
            pltpu.SemaphoreType.DMA,
        ],
    )
    def k(dst, zeros, ones, out, dst_v, ones_v, deg_sh, sem0, sem1):
        c = lax.axis_index("c")
        s = lax.axis_index("s")
        w = s * _NUM_CORES + c
        pltpu.sync_copy(zeros.at[pl.ds(s * rpt, rpt)],
                        deg_sh.at[pl.ds(s * rpt, rpt)])
        pltpu.sync_copy(ones, ones_v)
        plsc.subcore_barrier()
        sems = (sem0, sem1)

        def fetch_idx(j, sl):
            pltpu.sync_copy(dst.at[w, pl.ds(j * _K, _K)],
                            dst_v.at[pl.ds(sl * _K, _K)])

        def scat(sl):
            return pltpu.async_copy(
                ones_v, deg_sh.at[dst_v.at[pl.ds(sl * _K, _K)]], sems[sl],
                add=True)

        def drain(sl):
            pltpu.make_async_copy(
                ones_v, deg_sh.at[dst_v.at[pl.ds(sl * _K, _K)]],
                sems[sl]).wait()

        fetch_idx(0, 0)
        scat(0)

        def body(i, carry):
            j1 = 2 * i + 1
            fetch_idx(j1, 1)
            scat(1)
            drain(0)

            @pl.when(j1 + 1 < nchunks)
            def _():
                fetch_idx(j1 + 1, 0)
                scat(0)

            drain(1)
            return carry

        lax.fori_loop(0, nchunks // 2, body, 0)
        plsc.subcore_barrier()
        pltpu.sync_copy(deg_sh.at[pl.ds(s * rpt, rpt)],
                        out.at[c, pl.ds(s * rpt, rpt)])

    return k


@functools.lru_cache(maxsize=None)
def _make_dense(NP: int, R: int, da: int, din: int, dout: int, dnext: int,
                pre: bool):
    grid = NP // R

    def body(*refs):
        it = iter(refs)
        a0, a1, degp, h = next(it), next(it), next(it), next(it)
        wl = next(it) if pre else None
        wr, b = next(it), next(it)
        wn = next(it) if dnext > 0 else None
        out = next(it)
        pout = next(it) if dnext > 0 else None
        dsum = degp[0] + degp[1]
        deg = jnp.maximum(dsum[:, 0:1], 1.0)
        mean = (a0[...] + a1[...]) / deg
        if pre:
            acc = jnp.dot(mean, wl[...], preferred_element_type=jnp.float32)
        else:
            acc = mean
        acc = acc + jnp.dot(h[...], wr[...],
                            preferred_element_type=jnp.float32)
        res = jnp.maximum(acc + b[...], 0.0)
        out[...] = res
        if dnext > 0:
            pout[...] = jnp.dot(res, wn[...],
                                preferred_element_type=jnp.float32)

    in_specs = [
        pl.BlockSpec((R, da), lambda i: (i, 0)),
        pl.BlockSpec((R, da), lambda i: (i, 0)),
        pl.BlockSpec((2, R, 128), lambda i: (0, i, 0)),
        pl.BlockSpec((R, din), lambda i: (i, 0)),
    ]
    if pre:
        in_specs.append(pl.BlockSpec((da, dout), lambda i: (0, 0)))
    in_specs += [
        pl.BlockSpec((din, dout), lambda i: (0, 0)),
        pl.BlockSpec((1, dout), lambda i: (0, 0)),
    ]
    out_shape = [jax.ShapeDtypeStruct((NP, dout), jnp.float32)]
    out_specs = [pl.BlockSpec((R, dout), lambda i: (i, 0))]
    if dnext > 0:
        in_specs.append(pl.BlockSpec((dout, dnext), lambda i: (0, 0)))
        out_shape.append(jax.ShapeDtypeStruct((NP, dnext), jnp.float32))
        out_specs.append(pl.BlockSpec((R, dnext), lambda i: (i, 0)))

    return pl.pallas_call(
        body,
        grid=(grid,),
        in_specs=in_specs,
        out_specs=out_specs,
        out_shape=out_shape,
    )


@functools.lru_cache(maxsize=None)
def _make_final(NP: int, N: int, R: int, da: int, dain: int, dh: int):
    grid = NP // R

    def body(a0, a1, degp, h5, wr4, b4, pooled, logp, acc1, acc2):
        i = pl.program_id(0)

        @pl.when(i == 0)
        def _init():
            acc1[...] = jnp.zeros_like(acc1)
            acc2[...] = jnp.zeros_like(acc2)

        dsum = degp[0] + degp[1]
        deg = jnp.maximum(dsum[:, 0:1], 1.0)
        row = i * R + lax.broadcasted_iota(jnp.int32, (R, 1), 0)
        mask = (row < N).astype(jnp.float32)
        asum = (a0[...] + a1[...])[:, 0:da]
        contrib = asum / deg * mask
        acc1[...] += jnp.sum(contrib, axis=0, keepdims=True)
        acc2[...] += jnp.sum(h5[...] * mask, axis=0, keepdims=True)

        @pl.when(i == grid - 1)
        def _emit():
            s1 = acc1[...]
            s2 = acc2[...]
            p = (s1 + jnp.dot(s2, wr4[...],
                              preferred_element_type=jnp.float32)) / N
            p = p + b4[...]
            m = jnp.max(p, axis=1, keepdims=True)
            lse = jnp.log(jnp.sum(jnp.exp(p - m), axis=1, keepdims=True)) + m
            pooled[...] = p
            logp[...] = p - lse

    return pl.pallas_call(
        body,
        grid=(grid,),
        in_specs=[
            pl.BlockSpec((R, dain), lambda i: (i, 0)),
            pl.BlockSpec((R, dain), lambda i: (i, 0)),
            pl.BlockSpec((2, R, 128), lambda i: (0, i, 0)),
            pl.BlockSpec((R, dh), lambda i: (i, 0)),
            pl.BlockSpec((dh, da), lambda i: (0, 0)),
            pl.BlockSpec((1, da), lambda i: (0, 0)),
        ],
        out_specs=[
            pl.BlockSpec((1, da), lambda i: (0, 0)),
            pl.BlockSpec((1, da), lambda i: (0, 0)),
        ],
        out_shape=[
            jax.ShapeDtypeStruct((1, da), jnp.float32),
            jax.ShapeDtypeStruct((1, da), jnp.float32),
        ],
        scratch_shapes=[
            pltpu.VMEM((1, da), jnp.float32),
            pltpu.VMEM((1, dh), jnp.float32),
        ],
    )


def _round_up(v: int, m: int) -> int:
    return (v + m - 1) // m * m


def kernel(x, edge_index, Wl1, Wr1, b1, Wl2, Wr2, b2, Wl3, Wr3, b3,
           Wl4, Wr4, b4):
    N, D = x.shape
    E = edge_index.shape[1]
    H2 = Wl1.shape[1]
    H = Wl2.shape[1]
    C = Wl4.shape[1]

    NP = _round_up(N, _NUM_SUBCORES * 8)
    EP = _round_up(E, _NUM_WORKERS * _K * 2)
    R = NP // 8
    nchunks = EP // (_NUM_WORKERS * _K)

    pad = EP - E
    pad_idx = (N + (NP - N - 16)
               + (jnp.arange(pad, dtype=jnp.int32) % 16))
    src_p = jnp.concatenate([edge_index[0], pad_idx]).reshape(
        _NUM_WORKERS, EP // _NUM_WORKERS)
    dst_p = jnp.concatenate([edge_index[1], pad_idx]).reshape(
        _NUM_WORKERS, EP // _NUM_WORKERS)
    x_p = jnp.pad(x, ((0, NP - N), (0, 0)))

    zeros_d = jnp.zeros((NP, H), jnp.float32)
    ones_d = jnp.ones((_K, 128), jnp.float32)

    agg_h = _make_agg(NP, H, EP)
    deg_k = _make_deg(NP, EP)

    degp = deg_k(dst_p, zeros_d, ones_d)

    b1r = b1.reshape(1, -1)
    b2r = b2.reshape(1, -1)
    b3r = b3.reshape(1, -1)
    b4r = b4.reshape(1, -1)

    a1p = agg_h(x_p, src_p, dst_p, zeros_d)
    h1, p2 = _make_dense(NP, R, D, D, H2, H, True)(
        a1p[0], a1p[1], degp, x_p, Wl1, Wr1, b1r, Wl2)

    a2p = agg_h(p2, src_p, dst_p, zeros_d)
    h2 = _make_dense(NP, R, H, H2, H, 0, False)(
        a2p[0], a2p[1], degp, h1, Wr2, b2r)[0]

    a3p = agg_h(h2, src_p, dst_p, zeros_d)
    h3 = _make_dense(NP, R, H, H, H, 0, True)(
        a3p[0], a3p[1], degp, h2, Wl3, Wr3, b3r)[0]
    a4p = agg_h(h3, src_p, dst_p, zeros_d)
    h4 = _make_dense(NP, R, H, H, H, 0, True)(
        a4p[0], a4p[1], degp, h3, Wl3, Wr3, b3r)[0]
    a5p = agg_h(h4, src_p, dst_p, zeros_d)
    Wl4p = jnp.pad(Wl4, ((0, 0), (0, H - C)))
    h5, p6 = _make_dense(NP, R, H, H, H, H, True)(
        a5p[0], a5p[1], degp, h4, Wl3, Wr3, b3r, Wl4p)

    a6p = agg_h(p6, src_p, dst_p, zeros_d)
    pooled, logp = _make_final(NP, N, R, C, H, H)(
        a6p[0], a6p[1], degp, h5, Wr4, b4r)
    return (pooled, logp)
